# baseline (speedup 1.0000x reference)
import functools

import jax
import jax.numpy as jnp
from jax import lax
from jax.experimental import pallas as pl
from jax.experimental.pallas import tpu as pltpu
from jax.experimental.pallas import tpu_sc as plsc

HIDDEN = 1024
FFN = 1024
NUM_EXPERTS = 64
TOP_K = 2
TOKENS = 128

SC_NC = 2
SC_NS = 16
SC_LANES = 16
SC_WORKERS = SC_NC * SC_NS
TOK_PER_WORKER = TOKENS // SC_WORKERS
NCHUNK = NUM_EXPERTS // SC_LANES


def _gate_kernel(x_ref, gw_ref, logits_ref):
    logits_ref[...] = jax.lax.dot_general(
        x_ref[...], gw_ref[...], (((1,), (0,)), ((), ())),
        preferred_element_type=jnp.float32,
    )


def _sc_router(logits_hbm, comb_hbm, lrows, crows):
    wid = lax.axis_index("s") * SC_NC + lax.axis_index("c")
    base = wid * TOK_PER_WORKER
    pltpu.sync_copy(logits_hbm.at[pl.ds(base, TOK_PER_WORKER)], lrows)
    iota = lax.iota(jnp.int32, SC_LANES)
    bigi = jnp.full((SC_LANES,), jnp.int32(2 * NUM_EXPERTS), jnp.int32)
    negv = jnp.full((SC_LANES,), jnp.float32(-1e30), jnp.float32)
    zerov = jnp.zeros((SC_LANES,), jnp.float32)
    for t in range(TOK_PER_WORKER):
        chunks = [lrows[t, pl.ds(c * SC_LANES, SC_LANES)] for c in range(NCHUNK)]
        iotas = [iota + c * SC_LANES for c in range(NCHUNK)]
        m = chunks[0]
        for c in range(1, NCHUNK):
            m = jnp.maximum(m, chunks[c])
        l1 = jnp.max(m)
        i1 = jnp.int32(2 * NUM_EXPERTS)
        for c in range(NCHUNK):
            l1v = jnp.full((SC_LANES,), l1, jnp.float32)
            cand = jnp.min(jnp.where(chunks[c] == l1v, iotas[c], bigi))
            i1 = jnp.minimum(i1, cand)
        i1v = jnp.full((SC_LANES,), i1, jnp.int32)
        chunks2 = [
            jnp.where(iotas[c] == i1v, negv, chunks[c]) for c in range(NCHUNK)
        ]
        m2 = chunks2[0]
        for c in range(1, NCHUNK):
            m2 = jnp.maximum(m2, chunks2[c])
        l2 = jnp.max(m2)
        i2 = jnp.int32(2 * NUM_EXPERTS)
        for c in range(NCHUNK):
            l2v = jnp.full((SC_LANES,), l2, jnp.float32)
            cand = jnp.min(jnp.where(chunks2[c] == l2v, iotas[c], bigi))
            i2 = jnp.minimum(i2, cand)
        i2v = jnp.full((SC_LANES,), i2, jnp.int32)
        dv = jnp.full((SC_LANES,), l2 - l1, jnp.float32)
        ev = jnp.exp(dv)
        w1v = 1.0 / (1.0 + ev)
        w2v = 1.0 - w1v
        for c in range(NCHUNK):
            outv = jnp.where(iotas[c] == i1v, w1v, zerov)
            outv = jnp.where(iotas[c] == i2v, w2v, outv)
            crows[t, pl.ds(c * SC_LANES, SC_LANES)] = outv
    pltpu.sync_copy(crows, comb_hbm.at[pl.ds(base, TOK_PER_WORKER)])


_sc_router_call = functools.partial(
    pl.kernel,
    mesh=plsc.VectorSubcoreMesh(core_axis_name="c", subcore_axis_name="s"),
    out_type=jax.ShapeDtypeStruct((TOKENS, NUM_EXPERTS), jnp.float32),
    scratch_types=[
        pltpu.VMEM((TOK_PER_WORKER, NUM_EXPERTS), jnp.float32),
        pltpu.VMEM((TOK_PER_WORKER, NUM_EXPERTS), jnp.float32),
    ],
    compiler_params=pltpu.CompilerParams(needs_layout_passes=False),
)(_sc_router)


def _expert_kernel(x_ref, w1_ref, w2_ref, comb_ref, out_ref):
    e = pl.program_id(0)

    @pl.when(e == 0)
    def _():
        out_ref[...] = jnp.zeros_like(out_ref)

    x = x_ref[...].astype(jnp.bfloat16)
    w1e = w1_ref[0].astype(jnp.bfloat16)
    h = jax.lax.dot_general(
        x, w1e, (((1,), (1,)), ((), ())), preferred_element_type=jnp.float32
    )
    gate = h[:, :FFN]
    up = h[:, FFN:]
    act = gate * jax.lax.logistic(gate) * up
    w2e = w2_ref[0].astype(jnp.bfloat16)
    eo = jax.lax.dot_general(
        act.astype(jnp.bfloat16), w2e, (((1,), (1,)), ((), ())),
        preferred_element_type=jnp.float32,
    )
    comb = comb_ref[...]
    col = jax.lax.broadcasted_iota(jnp.int32, comb.shape, 1)
    cw = jnp.sum(jnp.where(col == e, comb, 0.0), axis=1, keepdims=True)
    out_ref[...] += cw * eo


@jax.jit
def kernel(hidden_states, gate_w, w1, w2):
    logits = pl.pallas_call(
        _gate_kernel,
        out_shape=jax.ShapeDtypeStruct((TOKENS, NUM_EXPERTS), jnp.float32),
    )(hidden_states, gate_w)

    comb = _sc_router_call(logits)

    out = pl.pallas_call(
        _expert_kernel,
        grid=(NUM_EXPERTS,),
        in_specs=[
            pl.BlockSpec((TOKENS, HIDDEN), lambda e: (0, 0)),
            pl.BlockSpec((1, 2 * FFN, HIDDEN), lambda e: (e, 0, 0)),
            pl.BlockSpec((1, HIDDEN, FFN), lambda e: (e, 0, 0)),
            pl.BlockSpec((TOKENS, NUM_EXPERTS), lambda e: (0, 0)),
        ],
        out_specs=pl.BlockSpec((TOKENS, HIDDEN), lambda e: (0, 0)),
        out_shape=jax.ShapeDtypeStruct((TOKENS, HIDDEN), jnp.float32),
    )(hidden_states, w1, w2, comb)

    return (out, logits)


# speedup vs baseline: 1.0739x; 1.0739x over previous
import functools

import jax
import jax.numpy as jnp
from jax.experimental import pallas as pl
from jax.experimental.pallas import tpu as pltpu

HIDDEN = 1024
FFN = 1024
NUM_EXPERTS = 64
TOP_K = 2
TOKENS = 128


def _fused_kernel(x_ref, gw_ref, w1_ref, w2_ref, out_ref, logits_ref, comb_ref):
    e = pl.program_id(0)

    @pl.when(e == 0)
    def _():
        logits = jax.lax.dot_general(
            x_ref[...], gw_ref[...], (((1,), (0,)), ((), ())),
            preferred_element_type=jnp.float32,
        )
        logits_ref[...] = logits
        m = jnp.max(logits, axis=-1, keepdims=True)
        ex = jnp.exp(logits - m)
        probs = ex / jnp.sum(ex, axis=-1, keepdims=True)
        col = jax.lax.broadcasted_iota(jnp.int32, probs.shape, 1)
        big = jnp.int32(NUM_EXPERTS)
        m1 = jnp.max(probs, axis=-1, keepdims=True)
        i1 = jnp.min(jnp.where(probs == m1, col, big), axis=-1, keepdims=True)
        oh1 = col == i1
        probs2 = jnp.where(oh1, -1.0, probs)
        m2 = jnp.max(probs2, axis=-1, keepdims=True)
        i2 = jnp.min(jnp.where(probs2 == m2, col, big), axis=-1, keepdims=True)
        oh2 = col == i2
        denom = m1 + m2
        comb_ref[...] = (jnp.where(oh1, m1, 0.0) + jnp.where(oh2, m2, 0.0)) / denom
        out_ref[...] = jnp.zeros_like(out_ref)

    x = x_ref[...].astype(jnp.bfloat16)
    w1e = w1_ref[0].astype(jnp.bfloat16)
    h = jax.lax.dot_general(
        x, w1e, (((1,), (1,)), ((), ())), preferred_element_type=jnp.float32
    )
    gate = h[:, :FFN]
    up = h[:, FFN:]
    act = gate * jax.lax.logistic(gate) * up
    w2e = w2_ref[0].astype(jnp.bfloat16)
    eo = jax.lax.dot_general(
        act.astype(jnp.bfloat16), w2e, (((1,), (1,)), ((), ())),
        preferred_element_type=jnp.float32,
    )
    comb = comb_ref[...]
    col = jax.lax.broadcasted_iota(jnp.int32, comb.shape, 1)
    cw = jnp.sum(jnp.where(col == e, comb, 0.0), axis=1, keepdims=True)
    out_ref[...] += cw * eo


@jax.jit
def kernel(hidden_states, gate_w, w1, w2):
    out, logits = pl.pallas_call(
        _fused_kernel,
        grid=(NUM_EXPERTS,),
        in_specs=[
            pl.BlockSpec((TOKENS, HIDDEN), lambda e: (0, 0)),
            pl.BlockSpec((HIDDEN, NUM_EXPERTS), lambda e: (0, 0)),
            pl.BlockSpec((1, 2 * FFN, HIDDEN), lambda e: (e, 0, 0)),
            pl.BlockSpec((1, HIDDEN, FFN), lambda e: (e, 0, 0)),
        ],
        out_specs=(
            pl.BlockSpec((TOKENS, HIDDEN), lambda e: (0, 0)),
            pl.BlockSpec((TOKENS, NUM_EXPERTS), lambda e: (0, 0)),
        ),
        out_shape=(
            jax.ShapeDtypeStruct((TOKENS, HIDDEN), jnp.float32),
            jax.ShapeDtypeStruct((TOKENS, NUM_EXPERTS), jnp.float32),
        ),
        scratch_shapes=[pltpu.VMEM((TOKENS, NUM_EXPERTS), jnp.float32)],
    )(hidden_states, gate_w, w1, w2)

    return (out, logits)
